# baseline (speedup 1.0000x reference)
import jax
import jax.numpy as jnp
from jax.experimental import pallas as pl
from jax.experimental.pallas import tpu as pltpu

_B = 1024
_Z = 128
_N = 64
_R = 4
_C = 16
_K = 512
_CB = 1024
_J = (_N * _N * _R) // _CB

_F32 = jnp.float32


def _group_softmax_lanes(x, g):
    e = jnp.exp(x)
    lane = jax.lax.broadcasted_iota(jnp.int32, x.shape, x.ndim - 1)
    size = x.shape[-1]
    s = e
    shift = 1
    while shift < g:
        left = pltpu.roll(s, size - shift, axis=x.ndim - 1)
        right = pltpu.roll(s, shift, axis=x.ndim - 1)
        s = s + jnp.where((lane & shift) == 0, left, right)
        shift *= 2
    return e / s


def _body(x_ref, w1_ref, b1_ref, w2_ref, b2_ref, w3_ref, b3_ref,
          wx_ref, bx_ref, wa_ref, ba_ref, xout_ref, aout_ref, h_ref):
    j = pl.program_id(0)

    @pl.when(j == 0)
    def _mlp_and_node_head():
        h = jnp.tanh(
            jnp.dot(x_ref[...], w1_ref[...], preferred_element_type=_F32)
            + b1_ref[...])
        h = jnp.tanh(
            jnp.dot(h, w2_ref[...], preferred_element_type=_F32)
            + b2_ref[...])
        h = jnp.tanh(
            jnp.dot(h, w3_ref[...], preferred_element_type=_F32)
            + b3_ref[...])
        h_ref[...] = h
        xl = jnp.dot(h, wx_ref[...], preferred_element_type=_F32) + bx_ref[...]
        xout_ref[...] = _group_softmax_lanes(xl, _C)

    al = (jnp.dot(h_ref[...], wa_ref[...], preferred_element_type=_F32)
          + ba_ref[...])
    aout_ref[...] = _group_softmax_lanes(al, _R)


def kernel(input, W1, b1, W2, b2, W3, b3, Wx, bx, Wa, ba):
    wa4 = Wa.reshape(_K, _N, _N, _R)
    wa_sym = (0.5 * (wa4 + wa4.transpose(0, 2, 1, 3))).reshape(_K, _N * _N * _R)
    ba4 = ba.reshape(_N, _N, _R)
    ba_sym = (0.5 * (ba4 + ba4.transpose(1, 0, 2))).reshape(1, _N * _N * _R)

    x_prob, adj = pl.pallas_call(
        _body,
        grid=(_J,),
        in_specs=[
            pl.BlockSpec((_B, _Z), lambda j: (0, 0)),
            pl.BlockSpec((_Z, 128), lambda j: (0, 0)),
            pl.BlockSpec((1, 128), lambda j: (0, 0)),
            pl.BlockSpec((128, 256), lambda j: (0, 0)),
            pl.BlockSpec((1, 256), lambda j: (0, 0)),
            pl.BlockSpec((256, _K), lambda j: (0, 0)),
            pl.BlockSpec((1, _K), lambda j: (0, 0)),
            pl.BlockSpec((_K, _N * _C), lambda j: (0, 0)),
            pl.BlockSpec((1, _N * _C), lambda j: (0, 0)),
            pl.BlockSpec((_K, _CB), lambda j: (0, j)),
            pl.BlockSpec((1, _CB), lambda j: (0, j)),
        ],
        out_specs=[
            pl.BlockSpec((_B, _N * _C), lambda j: (0, 0)),
            pl.BlockSpec((_B, _CB), lambda j: (0, j)),
        ],
        out_shape=[
            jax.ShapeDtypeStruct((_B, _N * _C), _F32),
            jax.ShapeDtypeStruct((_B, _N * _N * _R), _F32),
        ],
        scratch_shapes=[pltpu.VMEM((_B, _K), _F32)],
        compiler_params=pltpu.CompilerParams(
            dimension_semantics=("arbitrary",)),
    )(input, W1, b1.reshape(1, -1), W2, b2.reshape(1, -1),
      W3, b3.reshape(1, -1), Wx, bx.reshape(1, -1), wa_sym, ba_sym)

    return x_prob.reshape(_B, _N, _C), adj.reshape(_B, _N, _N, _R)


# speedup vs baseline: 1.0523x; 1.0523x over previous
import jax
import jax.numpy as jnp
from jax.experimental import pallas as pl
from jax.experimental.pallas import tpu as pltpu

_B = 1024
_Z = 128
_N = 64
_R = 4
_C = 16
_K = 512
_CB = 1024
_J = (_N * _N * _R) // _CB

_F32 = jnp.float32
_BF16 = jnp.bfloat16


def _group_softmax_lanes(x, g):
    e = jnp.exp(x)
    lane = jax.lax.broadcasted_iota(jnp.int32, x.shape, x.ndim - 1)
    size = x.shape[-1]
    s = e
    shift = 1
    while shift < g:
        left = pltpu.roll(s, size - shift, axis=x.ndim - 1)
        right = pltpu.roll(s, shift, axis=x.ndim - 1)
        s = s + jnp.where((lane & shift) == 0, left, right)
        shift *= 2
    return e / s


def _body(x_ref, w1_ref, b1_ref, w2_ref, b2_ref, w3_ref, b3_ref,
          wx_ref, bx_ref, wa_ref, ba_ref, xout_ref, aout_ref, h_ref):
    j = pl.program_id(0)

    @pl.when(j == 0)
    def _mlp_and_node_head():
        h = jnp.tanh(
            jnp.dot(x_ref[...], w1_ref[...], preferred_element_type=_F32)
            + b1_ref[...]).astype(_BF16)
        h = jnp.tanh(
            jnp.dot(h, w2_ref[...], preferred_element_type=_F32)
            + b2_ref[...]).astype(_BF16)
        h = jnp.tanh(
            jnp.dot(h, w3_ref[...], preferred_element_type=_F32)
            + b3_ref[...]).astype(_BF16)
        h_ref[...] = h
        xl = jnp.dot(h, wx_ref[...], preferred_element_type=_F32) + bx_ref[...]
        xout_ref[...] = _group_softmax_lanes(xl, _C)

    al = (jnp.dot(h_ref[...], wa_ref[...], preferred_element_type=_F32)
          + ba_ref[...])
    aout_ref[...] = _group_softmax_lanes(al, _R)


def kernel(input, W1, b1, W2, b2, W3, b3, Wx, bx, Wa, ba):
    wa4 = Wa.reshape(_K, _N, _N, _R)
    wa_sym = (0.5 * (wa4 + wa4.transpose(0, 2, 1, 3))).reshape(_K, _N * _N * _R)
    ba4 = ba.reshape(_N, _N, _R)
    ba_sym = (0.5 * (ba4 + ba4.transpose(1, 0, 2))).reshape(1, _N * _N * _R)

    x_prob, adj = pl.pallas_call(
        _body,
        grid=(_J,),
        in_specs=[
            pl.BlockSpec((_B, _Z), lambda j: (0, 0)),
            pl.BlockSpec((_Z, 128), lambda j: (0, 0)),
            pl.BlockSpec((1, 128), lambda j: (0, 0)),
            pl.BlockSpec((128, 256), lambda j: (0, 0)),
            pl.BlockSpec((1, 256), lambda j: (0, 0)),
            pl.BlockSpec((256, _K), lambda j: (0, 0)),
            pl.BlockSpec((1, _K), lambda j: (0, 0)),
            pl.BlockSpec((_K, _N * _C), lambda j: (0, 0)),
            pl.BlockSpec((1, _N * _C), lambda j: (0, 0)),
            pl.BlockSpec((_K, _CB), lambda j: (0, j)),
            pl.BlockSpec((1, _CB), lambda j: (0, j)),
        ],
        out_specs=[
            pl.BlockSpec((_B, _N * _C), lambda j: (0, 0)),
            pl.BlockSpec((_B, _CB), lambda j: (0, j)),
        ],
        out_shape=[
            jax.ShapeDtypeStruct((_B, _N * _C), _F32),
            jax.ShapeDtypeStruct((_B, _N * _N * _R), _F32),
        ],
        scratch_shapes=[pltpu.VMEM((_B, _K), _BF16)],
        compiler_params=pltpu.CompilerParams(
            dimension_semantics=("arbitrary",)),
    )(input.astype(_BF16), W1.astype(_BF16), b1.reshape(1, -1),
      W2.astype(_BF16), b2.reshape(1, -1), W3.astype(_BF16),
      b3.reshape(1, -1), Wx.astype(_BF16), bx.reshape(1, -1),
      wa_sym.astype(_BF16), ba_sym)

    return x_prob.reshape(_B, _N, _C), adj.reshape(_B, _N, _N, _R)


# speedup vs baseline: 1.1401x; 1.0834x over previous
import jax
import jax.numpy as jnp
from jax.experimental import pallas as pl
from jax.experimental.pallas import tpu as pltpu

_B = 1024
_Z = 128
_N = 64
_R = 4
_C = 16
_K = 512
_CB = 1024
_J = (_N * _N * _R) // _CB

_F32 = jnp.float32
_BF16 = jnp.bfloat16


def _group_softmax_lanes(x, g):
    e = jnp.exp(x)
    lane = jax.lax.broadcasted_iota(jnp.int32, x.shape, x.ndim - 1)
    size = x.shape[-1]
    s = e
    shift = 1
    while shift < g:
        left = pltpu.roll(s, size - shift, axis=x.ndim - 1)
        right = pltpu.roll(s, shift, axis=x.ndim - 1)
        s = s + jnp.where((lane & shift) == 0, left, right)
        shift *= 2
    return e / s


def _body(x_ref, w1_ref, b1_ref, w2_ref, b2_ref, w3_ref, b3_ref,
          wx_ref, bx_ref, wa_ref, ba_ref, gc_ref, ge_ref,
          xout_ref, aout_ref, h_ref):
    j = pl.program_id(0)

    @pl.when(j == 0)
    def _mlp_and_node_head():
        h = jnp.tanh(
            jnp.dot(x_ref[...], w1_ref[...], preferred_element_type=_F32)
            + b1_ref[...]).astype(_BF16)
        h = jnp.tanh(
            jnp.dot(h, w2_ref[...], preferred_element_type=_F32)
            + b2_ref[...]).astype(_BF16)
        h = jnp.tanh(
            jnp.dot(h, w3_ref[...], preferred_element_type=_F32)
            + b3_ref[...]).astype(_BF16)
        h_ref[...] = h
        xl = jnp.dot(h, wx_ref[...], preferred_element_type=_F32) + bx_ref[...]
        xout_ref[...] = _group_softmax_lanes(xl, _C)

    al = (jnp.dot(h_ref[...], wa_ref[...], preferred_element_type=_F32)
          + ba_ref[...])
    e = jnp.exp(al)
    dc = jnp.dot(e.astype(_BF16), gc_ref[...], preferred_element_type=_F32)
    rc = (1.0 / dc).astype(_BF16)
    rfull = jnp.dot(rc, ge_ref[...], preferred_element_type=_F32)
    aout_ref[...] = e * rfull


def kernel(input, W1, b1, W2, b2, W3, b3, Wx, bx, Wa, ba):
    wa4 = Wa.reshape(_K, _N, _N, _R)
    wa_sym = (0.5 * (wa4 + wa4.transpose(0, 2, 1, 3))).reshape(_K, _N * _N * _R)
    ba4 = ba.reshape(_N, _N, _R)
    ba_sym = (0.5 * (ba4 + ba4.transpose(1, 0, 2))).reshape(1, _N * _N * _R)

    lanes = jnp.arange(_CB)
    groups = jnp.arange(_CB // _R)
    gc = (lanes[:, None] // _R == groups[None, :]).astype(_BF16)
    ge = (groups[:, None] == lanes[None, :] // _R).astype(_BF16)

    x_prob, adj = pl.pallas_call(
        _body,
        grid=(_J,),
        in_specs=[
            pl.BlockSpec((_B, _Z), lambda j: (0, 0)),
            pl.BlockSpec((_Z, 128), lambda j: (0, 0)),
            pl.BlockSpec((1, 128), lambda j: (0, 0)),
            pl.BlockSpec((128, 256), lambda j: (0, 0)),
            pl.BlockSpec((1, 256), lambda j: (0, 0)),
            pl.BlockSpec((256, _K), lambda j: (0, 0)),
            pl.BlockSpec((1, _K), lambda j: (0, 0)),
            pl.BlockSpec((_K, _N * _C), lambda j: (0, 0)),
            pl.BlockSpec((1, _N * _C), lambda j: (0, 0)),
            pl.BlockSpec((_K, _CB), lambda j: (0, j)),
            pl.BlockSpec((1, _CB), lambda j: (0, j)),
            pl.BlockSpec((_CB, _CB // _R), lambda j: (0, 0)),
            pl.BlockSpec((_CB // _R, _CB), lambda j: (0, 0)),
        ],
        out_specs=[
            pl.BlockSpec((_B, _N * _C), lambda j: (0, 0)),
            pl.BlockSpec((_B, _CB), lambda j: (0, j)),
        ],
        out_shape=[
            jax.ShapeDtypeStruct((_B, _N * _C), _F32),
            jax.ShapeDtypeStruct((_B, _N * _N * _R), _F32),
        ],
        scratch_shapes=[pltpu.VMEM((_B, _K), _BF16)],
        compiler_params=pltpu.CompilerParams(
            dimension_semantics=("arbitrary",)),
    )(input.astype(_BF16), W1.astype(_BF16), b1.reshape(1, -1),
      W2.astype(_BF16), b2.reshape(1, -1), W3.astype(_BF16),
      b3.reshape(1, -1), Wx.astype(_BF16), bx.reshape(1, -1),
      wa_sym.astype(_BF16), ba_sym, gc, ge)

    return x_prob.reshape(_B, _N, _C), adj.reshape(_B, _N, _N, _R)


# speedup vs baseline: 1.2010x; 1.0534x over previous
import jax
import jax.numpy as jnp
from jax.experimental import pallas as pl
from jax.experimental.pallas import tpu as pltpu

_B = 1024
_Z = 128
_N = 64
_R = 4
_C = 16
_K = 512
_CB = 1024
_J = (_N * _N * _R) // _CB

_F32 = jnp.float32
_BF16 = jnp.bfloat16


def _group_softmax_mxu(logits, gc, ge):
    e = jnp.exp(logits)
    dc = jnp.dot(e.astype(_BF16), gc, preferred_element_type=_F32)
    rc = (1.0 / dc).astype(_BF16)
    rfull = jnp.dot(rc, ge, preferred_element_type=_F32)
    return e * rfull


def _body(x_ref, w1_ref, b1_ref, w2_ref, b2_ref, w3_ref, b3_ref,
          wx_ref, bx_ref, wa_ref, ba_ref, gc_ref, ge_ref,
          gxc_ref, gxe_ref, xout_ref, aout_ref, h_ref):
    j = pl.program_id(0)

    @pl.when(j == 0)
    def _mlp_and_node_head():
        h = jnp.tanh(
            jnp.dot(x_ref[...], w1_ref[...], preferred_element_type=_F32)
            + b1_ref[...]).astype(_BF16)
        h = jnp.tanh(
            jnp.dot(h, w2_ref[...], preferred_element_type=_F32)
            + b2_ref[...]).astype(_BF16)
        h = jnp.tanh(
            jnp.dot(h, w3_ref[...], preferred_element_type=_F32)
            + b3_ref[...]).astype(_BF16)
        h_ref[...] = h
        xl = jnp.dot(h, wx_ref[...], preferred_element_type=_F32) + bx_ref[...]
        xout_ref[...] = _group_softmax_mxu(xl, gxc_ref[...], gxe_ref[...])

    al = (jnp.dot(h_ref[...], wa_ref[...], preferred_element_type=_F32)
          + ba_ref[...])
    aout_ref[...] = _group_softmax_mxu(al, gc_ref[...], ge_ref[...])


def kernel(input, W1, b1, W2, b2, W3, b3, Wx, bx, Wa, ba):
    wa4 = Wa.astype(_BF16).reshape(_K, _N, _N, _R)
    wa_sym = (0.5 * (wa4 + wa4.transpose(0, 2, 1, 3))).reshape(
        _K, _N * _N * _R)
    ba4 = ba.reshape(_N, _N, _R)
    ba_sym = (0.5 * (ba4 + ba4.transpose(1, 0, 2))).reshape(1, _N * _N * _R)

    lanes = jnp.arange(_CB)
    groups = jnp.arange(_CB // _R)
    gc = (lanes[:, None] // _R == groups[None, :]).astype(_BF16)
    ge = (groups[:, None] == lanes[None, :] // _R).astype(_BF16)
    xlanes = jnp.arange(_N * _C)
    xgroups = jnp.arange(_N)
    gxc = (xlanes[:, None] // _C == xgroups[None, :]).astype(_BF16)
    gxe = (xgroups[:, None] == xlanes[None, :] // _C).astype(_BF16)

    x_prob, adj = pl.pallas_call(
        _body,
        grid=(_J,),
        in_specs=[
            pl.BlockSpec((_B, _Z), lambda j: (0, 0)),
            pl.BlockSpec((_Z, 128), lambda j: (0, 0)),
            pl.BlockSpec((1, 128), lambda j: (0, 0)),
            pl.BlockSpec((128, 256), lambda j: (0, 0)),
            pl.BlockSpec((1, 256), lambda j: (0, 0)),
            pl.BlockSpec((256, _K), lambda j: (0, 0)),
            pl.BlockSpec((1, _K), lambda j: (0, 0)),
            pl.BlockSpec((_K, _N * _C), lambda j: (0, 0)),
            pl.BlockSpec((1, _N * _C), lambda j: (0, 0)),
            pl.BlockSpec((_K, _CB), lambda j: (0, j)),
            pl.BlockSpec((1, _CB), lambda j: (0, j)),
            pl.BlockSpec((_CB, _CB // _R), lambda j: (0, 0)),
            pl.BlockSpec((_CB // _R, _CB), lambda j: (0, 0)),
            pl.BlockSpec((_N * _C, _N), lambda j: (0, 0)),
            pl.BlockSpec((_N, _N * _C), lambda j: (0, 0)),
        ],
        out_specs=[
            pl.BlockSpec((_B, _N * _C), lambda j: (0, 0)),
            pl.BlockSpec((_B, _CB), lambda j: (0, j)),
        ],
        out_shape=[
            jax.ShapeDtypeStruct((_B, _N * _C), _F32),
            jax.ShapeDtypeStruct((_B, _N * _N * _R), _F32),
        ],
        scratch_shapes=[pltpu.VMEM((_B, _K), _BF16)],
        compiler_params=pltpu.CompilerParams(
            dimension_semantics=("arbitrary",)),
    )(input.astype(_BF16), W1.astype(_BF16), b1.reshape(1, -1),
      W2.astype(_BF16), b2.reshape(1, -1), W3.astype(_BF16),
      b3.reshape(1, -1), Wx.astype(_BF16), bx.reshape(1, -1),
      wa_sym, ba_sym, gc, ge, gxc, gxe)

    return x_prob.reshape(_B, _N, _C), adj.reshape(_B, _N, _N, _R)
